# baseline (device time: 131017 ns/iter reference)
import jax
import jax.numpy as jnp
from jax import lax
from jax.experimental import pallas as pl
from jax.experimental.pallas import tpu as pltpu

N_Z = 4
N_Q = 4


def _q_to_xy(qq):
    xq = lax.div(qq, 2)
    yq = jnp.bitwise_xor(xq, lax.rem(qq, 2))
    return xq, yq


def kernel(O, Wo):
    B, S, H, D = O.shape
    K = H * D
    N = Wo.shape[1]
    s_out = S // N_Z
    n_strip = N // N_Q
    Bh = B // 2

    x_idx = lax.axis_index("x")
    y_idx = lax.axis_index("y")
    q_out = 2 * x_idx + jnp.bitwise_xor(x_idx, y_idx)
    Wq = lax.dynamic_slice(Wo, (0, q_out * n_strip), (K, n_strip))

    def body(o_ref, w_ref, out_ref, comm_ref, blocks_ref,
             send_semsA, recv_semsA, send_semsB, recv_semsB, credit_sem,
             send2r, recv2r, send2l, recv2l):
        my_x = lax.axis_index("x")
        my_y = lax.axis_index("y")
        my_z = lax.axis_index("z")
        q = 2 * my_x + jnp.bitwise_xor(my_x, my_y)

        zr = (my_x, my_y, lax.rem(my_z + 1, N_Z))
        zl = (my_x, my_y, lax.rem(my_z + N_Z - 1, N_Z))
        qr_x, qr_y = _q_to_xy(lax.rem(q + 1, N_Q))
        ql_x, ql_y = _q_to_xy(lax.rem(q + N_Q - 1, N_Q))
        qr = (qr_x, qr_y, my_z)
        ql = (ql_x, ql_y, my_z)

        barrier_sem = pltpu.get_barrier_semaphore()
        for nbr in (zl, zr, ql, qr):
            pl.semaphore_signal(
                barrier_sem, inc=1,
                device_id=nbr, device_id_type=pl.DeviceIdType.MESH,
            )
        pl.semaphore_wait(barrier_sem, 4)

        def partial_b(c, b_lo, b_len):
            o = o_ref[pl.ds(b_lo, b_len), pl.ds(c * s_out, s_out), :, :]
            acc = None
            for h in range(H):
                t = lax.dot_general(
                    o[:, :, h, :], w_ref[h * D:(h + 1) * D, :],
                    dimension_numbers=(((2,), (0,)), ((), ())),
                    preferred_element_type=jnp.float32,
                )
                acc = t if acc is None else acc + t
            return acc

        def partial_chunk(c):
            return partial_b(c, 0, B)

        def p1_copy(src_slot, dst_slot, b_lo, b_len, sems_s, sems_r, h):
            return pltpu.make_async_remote_copy(
                src_ref=comm_ref.at[src_slot, pl.ds(b_lo, b_len)],
                dst_ref=comm_ref.at[dst_slot, pl.ds(b_lo, b_len)],
                send_sem=sems_s.at[h],
                recv_sem=sems_r.at[h],
                device_id=zr,
                device_id_type=pl.DeviceIdType.MESH,
            )

        def p2_copy(s, b_lo, b_len, dev, sems_s, sems_r, k):
            return pltpu.make_async_remote_copy(
                src_ref=blocks_ref.at[s, pl.ds(b_lo, b_len)],
                dst_ref=blocks_ref.at[s, pl.ds(b_lo, b_len)],
                send_sem=sems_s.at[k],
                recv_sem=sems_r.at[k],
                device_id=dev,
                device_id_type=pl.DeviceIdType.MESH,
            )

        c0 = lax.rem(my_z + N_Z - 1, N_Z)
        c1 = lax.rem(my_z + N_Z - 2, N_Z)
        c2 = lax.rem(my_z + 1, N_Z)

        comm_ref[0, pl.ds(0, Bh)] = partial_b(c0, 0, Bh)
        h0A = p1_copy(0, 1, 0, Bh, send_semsA, recv_semsA, 0)
        h0A.start()
        comm_ref[0, pl.ds(Bh, Bh)] = partial_b(c0, Bh, Bh)
        h0B = p1_copy(0, 1, Bh, Bh, send_semsB, recv_semsB, 0)
        h0B.start()

        p1v = partial_chunk(c1)
        h0A.wait_recv()
        h0B.wait_recv()
        comm_ref[1] = comm_ref[1] + p1v
        h0A.wait_send()
        h0B.wait_send()
        h1 = p1_copy(1, 0, 0, B, send_semsA, recv_semsA, 1)
        h1.start()

        p2v = partial_chunk(c2)
        h1.wait_recv()
        h1.wait_send()
        pl.semaphore_signal(
            credit_sem, inc=1,
            device_id=zl, device_id_type=pl.DeviceIdType.MESH,
        )
        comm_ref[0, pl.ds(0, Bh)] = comm_ref[0, pl.ds(0, Bh)] + p2v[0:Bh]
        pl.semaphore_wait(credit_sem, 1)
        h2A = p1_copy(0, 1, 0, Bh, send_semsA, recv_semsA, 2)
        h2A.start()
        comm_ref[0, pl.ds(Bh, Bh)] = comm_ref[0, pl.ds(Bh, Bh)] + p2v[Bh:B]
        h2B = p1_copy(0, 1, Bh, Bh, send_semsB, recv_semsB, 2)
        h2B.start()

        sA = lax.rem(q + N_Q - 1, N_Q)
        sB = lax.rem(q + 1, N_Q)
        r_s1A = p2_copy(q, 0, Bh, qr, send2r, recv2r, 0)
        r_s1B = p2_copy(q, Bh, Bh, qr, send2r, recv2r, 1)
        r_s2 = p2_copy(sA, 0, Bh, qr, send2r, recv2r, 2)
        l_s1A = p2_copy(q, 0, Bh, ql, send2l, recv2l, 0)
        l_s1B = p2_copy(q, Bh, Bh, ql, send2l, recv2l, 1)
        l_s2 = p2_copy(sB, Bh, Bh, ql, send2l, recv2l, 2)

        pF = partial_chunk(my_z)
        h2A.wait_recv()
        blocks_ref[q, pl.ds(0, Bh)] = comm_ref[1, pl.ds(0, Bh)] + pF[0:Bh]
        r_s1A.start()
        l_s1A.start()
        h2B.wait_recv()
        blocks_ref[q, pl.ds(Bh, Bh)] = comm_ref[1, pl.ds(Bh, Bh)] + pF[Bh:B]
        r_s1B.start()
        l_s1B.start()
        h2A.wait_send()
        h2B.wait_send()

        r_s1A.wait_recv()
        r_s2.start()
        l_s1B.wait_recv()
        l_s2.start()
        r_s1B.wait_recv()
        l_s1A.wait_recv()
        r_s2.wait_recv()
        l_s2.wait_recv()

        for o in range(N_Q):
            out_ref[:, :, o * n_strip:(o + 1) * n_strip] = blocks_ref[o]

        r_s1A.wait_send()
        r_s1B.wait_send()
        r_s2.wait_send()
        l_s1A.wait_send()
        l_s1B.wait_send()
        l_s2.wait_send()

    return pl.pallas_call(
        body,
        out_shape=jax.ShapeDtypeStruct((B, s_out, N), jnp.float32),
        in_specs=[
            pl.BlockSpec(memory_space=pltpu.VMEM),
            pl.BlockSpec(memory_space=pltpu.VMEM),
        ],
        out_specs=pl.BlockSpec(memory_space=pltpu.VMEM),
        scratch_shapes=[
            pltpu.VMEM((2, B, s_out, n_strip), jnp.float32),
            pltpu.VMEM((N_Q, B, s_out, n_strip), jnp.float32),
            pltpu.SemaphoreType.DMA((N_Z - 1,)),
            pltpu.SemaphoreType.DMA((N_Z - 1,)),
            pltpu.SemaphoreType.DMA((N_Z - 1,)),
            pltpu.SemaphoreType.DMA((N_Z - 1,)),
            pltpu.SemaphoreType.REGULAR,
            pltpu.SemaphoreType.DMA((3,)),
            pltpu.SemaphoreType.DMA((3,)),
            pltpu.SemaphoreType.DMA((3,)),
            pltpu.SemaphoreType.DMA((3,)),
        ],
        compiler_params=pltpu.CompilerParams(collective_id=0),
    )(O, Wq)


# device time: 104928 ns/iter; 1.2486x vs baseline; 1.2486x over previous
import jax
import jax.numpy as jnp
from jax import lax
from jax.experimental import pallas as pl
from jax.experimental.pallas import tpu as pltpu

N_Z = 4
N_Q = 4


def _q_to_xy(qq):
    xq = lax.div(qq, 2)
    yq = jnp.bitwise_xor(xq, lax.rem(qq, 2))
    return xq, yq


def kernel(O, Wo):
    B, S, H, D = O.shape
    K = H * D
    N = Wo.shape[1]
    s_out = S // N_Z
    n_strip = N // N_Q

    O3 = O.reshape(B, S, K)

    x_idx = lax.axis_index("x")
    y_idx = lax.axis_index("y")
    q_out = 2 * x_idx + jnp.bitwise_xor(x_idx, y_idx)
    Wq = lax.dynamic_slice(Wo, (0, q_out * n_strip), (K, n_strip))

    def body(o_ref, w_ref, out_ref, comm_ref, blocks_ref,
             p1_send, p1_recv, credit_sem,
             s1r_send, s1r_recv, s2r_send, s2r_recv,
             s1l_send, s1l_recv, s2l_send, s2l_recv):
        my_x = lax.axis_index("x")
        my_y = lax.axis_index("y")
        my_z = lax.axis_index("z")
        q = 2 * my_x + jnp.bitwise_xor(my_x, my_y)

        zr = (my_x, my_y, lax.rem(my_z + 1, N_Z))
        zl = (my_x, my_y, lax.rem(my_z + N_Z - 1, N_Z))
        qr_x, qr_y = _q_to_xy(lax.rem(q + 1, N_Q))
        ql_x, ql_y = _q_to_xy(lax.rem(q + N_Q - 1, N_Q))
        qr = (qr_x, qr_y, my_z)
        ql = (ql_x, ql_y, my_z)

        barrier_sem = pltpu.get_barrier_semaphore()
        for nbr in (zl, zr, ql, qr):
            pl.semaphore_signal(
                barrier_sem, inc=1,
                device_id=nbr, device_id_type=pl.DeviceIdType.MESH,
            )
        pl.semaphore_wait(barrier_sem, 4)

        def partial_q(c, k):
            o = o_ref[pl.ds(k, 1), pl.ds(c * s_out, s_out), :]
            return lax.dot_general(
                o, w_ref[:, :],
                dimension_numbers=(((2,), (0,)), ((), ())),
                preferred_element_type=jnp.float32,
            )

        def p1_copy(h, k):
            return pltpu.make_async_remote_copy(
                src_ref=comm_ref.at[h % 2, pl.ds(k, 1)],
                dst_ref=comm_ref.at[(h + 1) % 2, pl.ds(k, 1)],
                send_sem=p1_send.at[h * B + k],
                recv_sem=p1_recv.at[h * B + k],
                device_id=zr,
                device_id_type=pl.DeviceIdType.MESH,
            )

        def p2_copy(s, k, dev, sems_s, sems_r, idx):
            return pltpu.make_async_remote_copy(
                src_ref=blocks_ref.at[s, pl.ds(k, 1)],
                dst_ref=blocks_ref.at[s, pl.ds(k, 1)],
                send_sem=sems_s.at[idx],
                recv_sem=sems_r.at[idx],
                device_id=dev,
                device_id_type=pl.DeviceIdType.MESH,
            )

        c0 = lax.rem(my_z + N_Z - 1, N_Z)
        c1 = lax.rem(my_z + N_Z - 2, N_Z)
        c2 = lax.rem(my_z + 1, N_Z)

        h0 = [p1_copy(0, k) for k in range(B)]
        h1 = [p1_copy(1, k) for k in range(B)]
        h2 = [p1_copy(2, k) for k in range(B)]

        for k in range(B):
            comm_ref[0, pl.ds(k, 1)] = partial_q(c0, k)
            h0[k].start()

        for k in range(B):
            p = partial_q(c1, k)
            h0[k].wait_recv()
            comm_ref[1, pl.ds(k, 1)] = comm_ref[1, pl.ds(k, 1)] + p
            h0[k].wait_send()
            h1[k].start()

        for k in range(B):
            p = partial_q(c2, k)
            h1[k].wait_recv()
            comm_ref[0, pl.ds(k, 1)] = comm_ref[0, pl.ds(k, 1)] + p
            h1[k].wait_send()
            pl.semaphore_signal(
                credit_sem, inc=1,
                device_id=zl, device_id_type=pl.DeviceIdType.MESH,
            )
            pl.semaphore_wait(credit_sem, 1)
            h2[k].start()

        sA = lax.rem(q + N_Q - 1, N_Q)
        sB = lax.rem(q + 1, N_Q)
        r_s1 = [p2_copy(q, k, qr, s1r_send, s1r_recv, k) for k in range(B)]
        l_s1 = [p2_copy(q, k, ql, s1l_send, s1l_recv, k) for k in range(B)]
        r_s2 = [p2_copy(sA, j, qr, s2r_send, s2r_recv, j) for j in range(2)]
        l_s2 = [p2_copy(sB, j, ql, s2l_send, s2l_recv, j - 2) for j in range(2, 4)]

        for k in range(B):
            p = partial_q(my_z, k)
            h2[k].wait_recv()
            blocks_ref[q, pl.ds(k, 1)] = comm_ref[1, pl.ds(k, 1)] + p
            r_s1[k].start()
            l_s1[k].start()

        r_s1[0].wait_recv()
        r_s2[0].start()
        r_s1[1].wait_recv()
        r_s2[1].start()
        l_s1[2].wait_recv()
        l_s2[0].start()
        l_s1[3].wait_recv()
        l_s2[1].start()

        r_s1[2].wait_recv()
        r_s1[3].wait_recv()
        l_s1[0].wait_recv()
        l_s1[1].wait_recv()
        for j in range(2):
            r_s2[j].wait_recv()
            l_s2[j].wait_recv()

        for o in range(N_Q):
            out_ref[:, :, o * n_strip:(o + 1) * n_strip] = blocks_ref[o]

        for k in range(B):
            h2[k].wait_send()
            r_s1[k].wait_send()
            l_s1[k].wait_send()
        for j in range(2):
            r_s2[j].wait_send()
            l_s2[j].wait_send()

    return pl.pallas_call(
        body,
        out_shape=jax.ShapeDtypeStruct((B, s_out, N), jnp.float32),
        in_specs=[
            pl.BlockSpec(memory_space=pltpu.VMEM),
            pl.BlockSpec(memory_space=pltpu.VMEM),
        ],
        out_specs=pl.BlockSpec(memory_space=pltpu.VMEM),
        scratch_shapes=[
            pltpu.VMEM((2, B, s_out, n_strip), jnp.float32),
            pltpu.VMEM((N_Q, B, s_out, n_strip), jnp.float32),
            pltpu.SemaphoreType.DMA(((N_Z - 1) * B,)),
            pltpu.SemaphoreType.DMA(((N_Z - 1) * B,)),
            pltpu.SemaphoreType.REGULAR,
            pltpu.SemaphoreType.DMA((B,)),
            pltpu.SemaphoreType.DMA((B,)),
            pltpu.SemaphoreType.DMA((2,)),
            pltpu.SemaphoreType.DMA((2,)),
            pltpu.SemaphoreType.DMA((B,)),
            pltpu.SemaphoreType.DMA((B,)),
            pltpu.SemaphoreType.DMA((2,)),
            pltpu.SemaphoreType.DMA((2,)),
        ],
        compiler_params=pltpu.CompilerParams(collective_id=0),
    )(O3, Wq)


# device time: 101586 ns/iter; 1.2897x vs baseline; 1.0329x over previous
import jax
import jax.numpy as jnp
from jax import lax
from jax.experimental import pallas as pl
from jax.experimental.pallas import tpu as pltpu

N_Z = 4
N_Q = 4


def _q_to_xy(qq):
    xq = lax.div(qq, 2)
    yq = jnp.bitwise_xor(xq, lax.rem(qq, 2))
    return xq, yq


def kernel(O, Wo):
    B, S, H, D = O.shape
    K = H * D
    N = Wo.shape[1]
    s_out = S // N_Z
    n_strip = N // N_Q

    OT = jnp.transpose(O.reshape(B, S, K), (0, 2, 1))

    x_idx = lax.axis_index("x")
    y_idx = lax.axis_index("y")
    q_out = 2 * x_idx + jnp.bitwise_xor(x_idx, y_idx)
    Wq = lax.dynamic_slice(Wo, (0, q_out * n_strip), (K, n_strip))

    def body(o_ref, w_ref, out_ref, comm_ref, blocks_ref,
             p1_send, p1_recv, credit_sem,
             s1r_send, s1r_recv, s2r_send, s2r_recv,
             s1l_send, s1l_recv, s2l_send, s2l_recv):
        my_x = lax.axis_index("x")
        my_y = lax.axis_index("y")
        my_z = lax.axis_index("z")
        q = 2 * my_x + jnp.bitwise_xor(my_x, my_y)

        zr = (my_x, my_y, lax.rem(my_z + 1, N_Z))
        zl = (my_x, my_y, lax.rem(my_z + N_Z - 1, N_Z))
        qr_x, qr_y = _q_to_xy(lax.rem(q + 1, N_Q))
        ql_x, ql_y = _q_to_xy(lax.rem(q + N_Q - 1, N_Q))
        qr = (qr_x, qr_y, my_z)
        ql = (ql_x, ql_y, my_z)

        barrier_sem = pltpu.get_barrier_semaphore()
        for nbr in (zl, zr, ql, qr):
            pl.semaphore_signal(
                barrier_sem, inc=1,
                device_id=nbr, device_id_type=pl.DeviceIdType.MESH,
            )
        pl.semaphore_wait(barrier_sem, 4)

        def partial_q(c, k):
            o = o_ref[pl.ds(k, 1), :, pl.ds(c * s_out, s_out)]
            return lax.dot_general(
                o, w_ref[:, :],
                dimension_numbers=(((1,), (0,)), ((), ())),
                preferred_element_type=jnp.float32,
            )

        def p1_copy(h, k):
            return pltpu.make_async_remote_copy(
                src_ref=comm_ref.at[h % 2, pl.ds(k, 1)],
                dst_ref=comm_ref.at[(h + 1) % 2, pl.ds(k, 1)],
                send_sem=p1_send.at[h * B + k],
                recv_sem=p1_recv.at[h * B + k],
                device_id=zr,
                device_id_type=pl.DeviceIdType.MESH,
            )

        def p2_copy(s, k, dev, sems_s, sems_r, idx):
            return pltpu.make_async_remote_copy(
                src_ref=blocks_ref.at[s, pl.ds(k, 1)],
                dst_ref=blocks_ref.at[s, pl.ds(k, 1)],
                send_sem=sems_s.at[idx],
                recv_sem=sems_r.at[idx],
                device_id=dev,
                device_id_type=pl.DeviceIdType.MESH,
            )

        c0 = lax.rem(my_z + N_Z - 1, N_Z)
        c1 = lax.rem(my_z + N_Z - 2, N_Z)
        c2 = lax.rem(my_z + 1, N_Z)

        h0 = [p1_copy(0, k) for k in range(B)]
        h1 = [p1_copy(1, k) for k in range(B)]
        h2 = [p1_copy(2, k) for k in range(B)]

        for k in range(B):
            comm_ref[0, pl.ds(k, 1)] = partial_q(c0, k)
            h0[k].start()

        for k in range(B):
            p = partial_q(c1, k)
            h0[k].wait_recv()
            comm_ref[1, pl.ds(k, 1)] = comm_ref[1, pl.ds(k, 1)] + p
            h0[k].wait_send()
            h1[k].start()

        for k in range(B):
            p = partial_q(c2, k)
            h1[k].wait_recv()
            comm_ref[0, pl.ds(k, 1)] = comm_ref[0, pl.ds(k, 1)] + p
            h1[k].wait_send()
            pl.semaphore_signal(
                credit_sem, inc=1,
                device_id=zl, device_id_type=pl.DeviceIdType.MESH,
            )
            pl.semaphore_wait(credit_sem, 1)
            h2[k].start()

        sA = lax.rem(q + N_Q - 1, N_Q)
        sB = lax.rem(q + 1, N_Q)
        r_s1 = [p2_copy(q, k, qr, s1r_send, s1r_recv, k) for k in range(B)]
        l_s1 = [p2_copy(q, k, ql, s1l_send, s1l_recv, k) for k in range(B)]
        r_s2 = [p2_copy(sA, j, qr, s2r_send, s2r_recv, j) for j in range(2)]
        l_s2 = [p2_copy(sB, j, ql, s2l_send, s2l_recv, j - 2) for j in range(2, 4)]

        for k in range(B):
            p = partial_q(my_z, k)
            h2[k].wait_recv()
            blocks_ref[q, pl.ds(k, 1)] = comm_ref[1, pl.ds(k, 1)] + p
            r_s1[k].start()
            l_s1[k].start()

        r_s1[0].wait_recv()
        r_s2[0].start()
        r_s1[1].wait_recv()
        r_s2[1].start()
        l_s1[2].wait_recv()
        l_s2[0].start()
        l_s1[3].wait_recv()
        l_s2[1].start()

        r_s1[2].wait_recv()
        r_s1[3].wait_recv()
        l_s1[0].wait_recv()
        l_s1[1].wait_recv()
        for j in range(2):
            r_s2[j].wait_recv()
            l_s2[j].wait_recv()

        for o in range(N_Q):
            out_ref[:, :, o * n_strip:(o + 1) * n_strip] = blocks_ref[o]

        for k in range(B):
            h2[k].wait_send()
            r_s1[k].wait_send()
            l_s1[k].wait_send()
        for j in range(2):
            r_s2[j].wait_send()
            l_s2[j].wait_send()

    return pl.pallas_call(
        body,
        out_shape=jax.ShapeDtypeStruct((B, s_out, N), jnp.float32),
        in_specs=[
            pl.BlockSpec(memory_space=pltpu.VMEM),
            pl.BlockSpec(memory_space=pltpu.VMEM),
        ],
        out_specs=pl.BlockSpec(memory_space=pltpu.VMEM),
        scratch_shapes=[
            pltpu.VMEM((2, B, s_out, n_strip), jnp.float32),
            pltpu.VMEM((N_Q, B, s_out, n_strip), jnp.float32),
            pltpu.SemaphoreType.DMA(((N_Z - 1) * B,)),
            pltpu.SemaphoreType.DMA(((N_Z - 1) * B,)),
            pltpu.SemaphoreType.REGULAR,
            pltpu.SemaphoreType.DMA((B,)),
            pltpu.SemaphoreType.DMA((B,)),
            pltpu.SemaphoreType.DMA((2,)),
            pltpu.SemaphoreType.DMA((2,)),
            pltpu.SemaphoreType.DMA((B,)),
            pltpu.SemaphoreType.DMA((B,)),
            pltpu.SemaphoreType.DMA((2,)),
            pltpu.SemaphoreType.DMA((2,)),
        ],
        compiler_params=pltpu.CompilerParams(collective_id=0),
    )(OT, Wq)
